# baseline (speedup 1.0000x reference)
import functools

import jax
import jax.numpy as jnp
from jax import lax
from jax.experimental import pallas as pl
from jax.experimental.pallas import tpu as pltpu
from jax.experimental.pallas import tpu_sc as plsc

BATCH = 16384
EMBED = 64
HIDDEN = 256

NUM_CORES = 2
NUM_SUBCORES = 16
NUM_WORKERS = NUM_CORES * NUM_SUBCORES
B_PER_W = BATCH // NUM_WORKERS
CHUNK = 256


def _make_gather_kernel():
    mesh = plsc.VectorSubcoreMesh(
        core_axis_name="c",
        subcore_axis_name="s",
        num_cores=NUM_CORES,
        num_subcores=NUM_SUBCORES,
    )
    out_type = (
        jax.ShapeDtypeStruct((BATCH, 2 * EMBED), jnp.float32),
        jax.ShapeDtypeStruct((BATCH, 2 * EMBED), jnp.float32),
    )

    @functools.partial(
        pl.kernel,
        mesh=mesh,
        out_type=out_type,
        scratch_types=[
            pltpu.VMEM((CHUNK,), jnp.int32),
            pltpu.VMEM((CHUNK,), jnp.int32),
            pltpu.VMEM((CHUNK, 2 * EMBED), jnp.float32),
            pltpu.VMEM((CHUNK, 2 * EMBED), jnp.float32),
            pltpu.SemaphoreType.DMA,
            pltpu.SemaphoreType.DMA,
        ],
    )
    def gather_kernel(
        user_pairs_hbm,
        movie_pairs_hbm,
        users_hbm,
        movies_hbm,
        out_u_hbm,
        out_m_hbm,
        idx_u,
        idx_m,
        rows_u,
        rows_m,
        sem_u,
        sem_m,
    ):
        wid = lax.axis_index("s") * NUM_CORES + lax.axis_index("c")
        base = wid * B_PER_W
        for c in range(B_PER_W // CHUNK):
            base_c = base + c * CHUNK
            pltpu.sync_copy(users_hbm.at[pl.ds(base_c, CHUNK)], idx_u)
            pltpu.sync_copy(movies_hbm.at[pl.ds(base_c, CHUNK)], idx_m)
            cp_u = pltpu.async_copy(user_pairs_hbm.at[idx_u], rows_u, sem_u)
            cp_m = pltpu.async_copy(movie_pairs_hbm.at[idx_m], rows_m, sem_m)
            cp_u.wait()
            cp_m.wait()
            pltpu.sync_copy(rows_u, out_u_hbm.at[pl.ds(base_c, CHUNK)])
            pltpu.sync_copy(rows_m, out_m_hbm.at[pl.ds(base_c, CHUNK)])

    return gather_kernel


@functools.lru_cache(maxsize=1)
def _get_gather():
    return _make_gather_kernel()


def _mlp_body(
    up_ref, mp_ref, pu_ref, pm_ref, w1u_ref, w1m_ref, b1_ref, w2_ref, b2_ref, o_ref
):
    u = jnp.where(pu_ref[...] > 0, up_ref[:, EMBED:], up_ref[:, :EMBED])
    m = jnp.where(pm_ref[...] > 0, mp_ref[:, EMBED:], mp_ref[:, :EMBED])
    h = (
        jnp.dot(u, w1u_ref[...], preferred_element_type=jnp.float32)
        + jnp.dot(m, w1m_ref[...], preferred_element_type=jnp.float32)
        + b1_ref[...]
    )
    h = jnp.maximum(h, 0.0)
    o_ref[...] = jnp.sum(h * w2_ref[...], axis=1, keepdims=True) + b2_ref[...]


def _mlp(up, mp, pu, pm, w1u, w1m, b1_2d, w2, b2_2d, block_rows=2048):
    grid = (BATCH // block_rows,)
    return pl.pallas_call(
        _mlp_body,
        grid=grid,
        in_specs=[
            pl.BlockSpec((block_rows, 2 * EMBED), lambda i: (i, 0)),
            pl.BlockSpec((block_rows, 2 * EMBED), lambda i: (i, 0)),
            pl.BlockSpec((block_rows, 1), lambda i: (i, 0)),
            pl.BlockSpec((block_rows, 1), lambda i: (i, 0)),
            pl.BlockSpec((EMBED, HIDDEN), lambda i: (0, 0)),
            pl.BlockSpec((EMBED, HIDDEN), lambda i: (0, 0)),
            pl.BlockSpec((1, HIDDEN), lambda i: (0, 0)),
            pl.BlockSpec((1, HIDDEN), lambda i: (0, 0)),
            pl.BlockSpec((1, 1), lambda i: (0, 0)),
        ],
        out_specs=pl.BlockSpec((block_rows, 1), lambda i: (i, 0)),
        out_shape=jax.ShapeDtypeStruct((BATCH, 1), jnp.float32),
    )(up, mp, pu, pm, w1u, w1m, b1_2d, w2, b2_2d)


@jax.jit
def kernel(users, movies, user_table, movie_table, W1, b1, W2, b2):
    users = users.astype(jnp.int32)
    movies = movies.astype(jnp.int32)
    user_pairs = user_table.reshape(-1, 2 * EMBED)
    movie_pairs = movie_table.reshape(-1, 2 * EMBED)
    u_pair_idx = users >> 1
    m_pair_idx = movies >> 1
    pu = (users & 1).reshape(-1, 1)
    pm = (movies & 1).reshape(-1, 1)
    up_rows, mp_rows = _get_gather()(user_pairs, movie_pairs, u_pair_idx, m_pair_idx)
    w1t = W1.T
    w1u = w1t[:EMBED]
    w1m = w1t[EMBED:]
    b1_2d = b1.reshape(1, HIDDEN)
    b2_2d = b2.reshape(1, 1)
    return _mlp(up_rows, mp_rows, pu, pm, w1u, w1m, b1_2d, W2, b2_2d)
